# baseline (speedup 1.0000x reference)
import jax
import jax.numpy as jnp
from jax import lax
from jax.experimental import pallas as pl
from jax.experimental.pallas import tpu as pltpu
from jax.experimental.pallas import tpu_sc as plsc

B = 4096
S = 200
D = 64
NC = 2
NS = 16
NW = NC * NS
SEQ_PER_W = B // NW
QCHUNK = 4
CHUNK = QCHUNK * S
NCHUNK = SEQ_PER_W // QCHUNK
LANES = 16
DG = D // LANES


def _embed_body(idx_hbm, table_hbm, pos_hbm, out_hbm, idx_v, rows_v, pos_v, sem):
    wid = lax.axis_index("s") * NC + lax.axis_index("c")
    seq_base = wid * SEQ_PER_W
    pltpu.sync_copy(pos_hbm, pos_v)

    def chunk_body(g, carry):
        seq0 = seq_base + g * QCHUNK
        pltpu.sync_copy(idx_hbm.at[pl.ds(seq0, QCHUNK)], idx_v)
        for q in range(QCHUNK):
            pltpu.async_copy(
                table_hbm.at[idx_v.at[q]], rows_v.at[pl.ds(q * S, S)], sem
            )
        for q in range(QCHUNK):
            pltpu.make_async_copy(
                table_hbm.at[idx_v.at[q]], rows_v.at[pl.ds(q * S, S)], sem
            ).wait()

        def s_body(s, c):
            for d in range(DG):
                pv = pos_v[s, pl.ds(d * LANES, LANES)]
                for q in range(QCHUNK):
                    plsc.addupdate(rows_v.at[q * S + s, pl.ds(d * LANES, LANES)], pv)
            return c

        lax.fori_loop(0, S, s_body, 0)
        for q in range(QCHUNK):
            pltpu.sync_copy(rows_v.at[pl.ds(q * S, S)], out_hbm.at[seq0 + q])
        return carry

    lax.fori_loop(0, NCHUNK, chunk_body, 0)


def kernel(batch_seqs, item_emb, pos_weight):
    k = pl.kernel(
        _embed_body,
        out_type=jax.ShapeDtypeStruct((B, S, D), jnp.float32),
        mesh=plsc.VectorSubcoreMesh(core_axis_name="c", subcore_axis_name="s"),
        compiler_params=pltpu.CompilerParams(use_tc_tiling_on_sc=False),
        scratch_types=[
            pltpu.VMEM((QCHUNK, S), jnp.int32),
            pltpu.VMEM((CHUNK, D), jnp.float32),
            pltpu.VMEM((S, D), jnp.float32),
            pltpu.SemaphoreType.DMA,
        ],
    )
    return k(batch_seqs, item_emb, pos_weight)
